# baseline (speedup 1.0000x reference)
import math

import jax
import jax.numpy as jnp
from jax.experimental import pallas as pl
from jax.experimental.pallas import tpu as pltpu

_HALF_LOG_2PI = 0.5 * math.log(2.0 * math.pi)
_LOG2E = 1.4426950408889634
_LN2 = 0.6931471805599453
_CH = 128


def _mog_block(zf_ref, means_ref, logvars_ref, w_ref, out_ref):
    mu0 = means_ref[...]
    lv0 = logvars_ref[...]
    w = w_ref[...]
    K, L = mu0.shape
    R = zf_ref.shape[0] // 2

    wmax = jnp.max(w, axis=0, keepdims=True)
    logw = (w - wmax) - jnp.log(jnp.sum(jnp.exp(w - wmax), axis=0, keepdims=True))

    mu = jnp.concatenate([mu0, mu0], axis=1)
    lv = jnp.concatenate([lv0, lv0], axis=1)
    lw = jnp.concatenate([logw, logw], axis=1)

    A = -0.5 * jnp.exp(-lv)
    Bc = (-2.0 * A) * mu
    C = (lw - _HALF_LOG_2PI - 0.5 * lv) + A * mu * mu
    M = jnp.max(C, axis=0, keepdims=True)
    A2 = (A * _LOG2E).astype(jnp.bfloat16)
    B2 = (Bc * _LOG2E).astype(jnp.bfloat16)
    C2 = ((C - M) * _LOG2E).astype(jnp.bfloat16)

    smin = jnp.float32(jnp.inf)
    for c in range(R // _CH):
        rows = pl.ds(c * _CH, _CH)
        z = jnp.concatenate([zf_ref[rows, :], zf_ref[pl.ds(R + c * _CH, _CH), :]], axis=1).astype(jnp.bfloat16)
        z2 = z * z
        s = jnp.zeros(z.shape, jnp.bfloat16)
        for k in range(K):
            x = A2[k : k + 1, :] * z2 + B2[k : k + 1, :] * z + C2[k : k + 1, :]
            s = s + jnp.exp2(x)
        sf = s.astype(jnp.float32)
        smin = jnp.minimum(smin, jnp.min(sf))
        yt = (-(M + _LN2 * jnp.log2(sf))).T
        out_ref[:, pl.ds(c * _CH, _CH)] = yt[:L, :]
        out_ref[:, pl.ds(R + c * _CH, _CH)] = yt[L:, :]

    @pl.when(smin < 1e-30)
    def _fixup():
        for c in range(R // _CH):
            rows = pl.ds(c * _CH, _CH)
            z = jnp.concatenate([zf_ref[rows, :], zf_ref[pl.ds(R + c * _CH, _CH), :]], axis=1)
            z2 = z * z
            Af = A2.astype(jnp.float32)
            Bf = B2.astype(jnp.float32)
            Cf = C2.astype(jnp.float32)
            m = Af[0:1, :] * z2 + Bf[0:1, :] * z + Cf[0:1, :]
            for k in range(1, K):
                x = Af[k : k + 1, :] * z2 + Bf[k : k + 1, :] * z + Cf[k : k + 1, :]
                m = jnp.maximum(m, x)
            s2 = jnp.zeros(z.shape, jnp.float32)
            for k in range(K):
                x = Af[k : k + 1, :] * z2 + Bf[k : k + 1, :] * z + Cf[k : k + 1, :]
                s2 = s2 + jnp.exp2(x - m)
            yt2 = (-(M + _LN2 * (m + jnp.log2(s2)))).T
            out_ref[:, pl.ds(c * _CH, _CH)] = yt2[:L, :]
            out_ref[:, pl.ds(R + c * _CH, _CH)] = yt2[L:, :]


def kernel(z, means, logvars, w):
    B, L = z.shape
    K = means.shape[0]
    R = 2048
    nblk = B // (2 * R)
    w_b = jnp.broadcast_to(w.reshape(K, 1), (K, L))
    return pl.pallas_call(
        _mog_block,
        grid=(nblk,),
        in_specs=[
            pl.BlockSpec((2 * R, L), lambda i: (i, 0)),
            pl.BlockSpec((K, L), lambda i: (0, 0)),
            pl.BlockSpec((K, L), lambda i: (0, 0)),
            pl.BlockSpec((K, L), lambda i: (0, 0)),
        ],
        out_specs=pl.BlockSpec((L, 2 * R), lambda i: (0, i)),
        out_shape=jax.ShapeDtypeStruct((L, B), jnp.float32),
        compiler_params=pltpu.CompilerParams(
            dimension_semantics=("arbitrary",),
        ),
    )(z, means, logvars, w_b)


# speedup vs baseline: 1.0004x; 1.0004x over previous
import math

import jax
import jax.numpy as jnp
from jax.experimental import pallas as pl
from jax.experimental.pallas import tpu as pltpu

_HALF_LOG_2PI = 0.5 * math.log(2.0 * math.pi)
_LOG2E = 1.4426950408889634
_LN2 = 0.6931471805599453
_CH = 128


def _mog_block(zf_ref, means_ref, logvars_ref, w_ref, out_ref):
    mu0 = means_ref[...]
    lv0 = logvars_ref[...]
    w = w_ref[...]
    K, L = mu0.shape
    R = zf_ref.shape[0] // 2

    wmax = jnp.max(w, axis=0, keepdims=True)
    logw = (w - wmax) - jnp.log(jnp.sum(jnp.exp(w - wmax), axis=0, keepdims=True))

    mu = jnp.concatenate([mu0, mu0], axis=1)
    lv = jnp.concatenate([lv0, lv0], axis=1)
    lw = jnp.concatenate([logw, logw], axis=1)

    A = -0.5 * jnp.exp(-lv)
    Bc = (-2.0 * A) * mu
    C = (lw - _HALF_LOG_2PI - 0.5 * lv) + A * mu * mu
    M = jnp.max(C, axis=0, keepdims=True)
    A2 = (A * _LOG2E).astype(jnp.bfloat16)
    B2 = (Bc * _LOG2E).astype(jnp.bfloat16)
    C2 = ((C - M) * _LOG2E).astype(jnp.bfloat16)

    smin = jnp.float32(jnp.inf)
    for c in range(R // _CH):
        rows = pl.ds(c * _CH, _CH)
        z = jnp.concatenate([zf_ref[rows, :], zf_ref[pl.ds(R + c * _CH, _CH), :]], axis=1).astype(jnp.bfloat16)
        z2 = z * z
        s = jnp.zeros(z.shape, jnp.bfloat16)
        for k in range(K):
            x = A2[k : k + 1, :] * z2 + B2[k : k + 1, :] * z + C2[k : k + 1, :]
            s = s + jnp.exp2(x)
        sf = s.astype(jnp.float32)
        smin = jnp.minimum(smin, jnp.min(sf))
        yt = (-(M + _LN2 * jnp.log2(sf))).T
        out_ref[:, pl.ds(c * _CH, _CH)] = yt[:L, :]
        out_ref[:, pl.ds(R + c * _CH, _CH)] = yt[L:, :]

    @pl.when(smin < 1e-30)
    def _fixup():
        for c in range(R // _CH):
            rows = pl.ds(c * _CH, _CH)
            z = jnp.concatenate([zf_ref[rows, :], zf_ref[pl.ds(R + c * _CH, _CH), :]], axis=1)
            z2 = z * z
            Af = A * _LOG2E
            Bf = Bc * _LOG2E
            Cf = (C - M) * _LOG2E
            m = Af[0:1, :] * z2 + Bf[0:1, :] * z + Cf[0:1, :]
            for k in range(1, K):
                x = Af[k : k + 1, :] * z2 + Bf[k : k + 1, :] * z + Cf[k : k + 1, :]
                m = jnp.maximum(m, x)
            s2 = jnp.zeros(z.shape, jnp.float32)
            for k in range(K):
                x = Af[k : k + 1, :] * z2 + Bf[k : k + 1, :] * z + Cf[k : k + 1, :]
                s2 = s2 + jnp.exp2(x - m)
            yt2 = (-(M + _LN2 * (m + jnp.log2(s2)))).T
            out_ref[:, pl.ds(c * _CH, _CH)] = yt2[:L, :]
            out_ref[:, pl.ds(R + c * _CH, _CH)] = yt2[L:, :]


def kernel(z, means, logvars, w):
    B, L = z.shape
    K = means.shape[0]
    R = 2048
    nblk = B // (2 * R)
    w_b = jnp.broadcast_to(w.reshape(K, 1), (K, L))
    return pl.pallas_call(
        _mog_block,
        grid=(nblk,),
        in_specs=[
            pl.BlockSpec((2 * R, L), lambda i: (i, 0)),
            pl.BlockSpec((K, L), lambda i: (0, 0)),
            pl.BlockSpec((K, L), lambda i: (0, 0)),
            pl.BlockSpec((K, L), lambda i: (0, 0)),
        ],
        out_specs=pl.BlockSpec((L, 2 * R), lambda i: (0, i)),
        out_shape=jax.ShapeDtypeStruct((L, B), jnp.float32),
        compiler_params=pltpu.CompilerParams(
            dimension_semantics=("arbitrary",),
        ),
    )(z, means, logvars, w_b)
